# baseline (speedup 1.0000x reference)
import jax
import jax.numpy as jnp
from jax import lax
from jax.experimental import pallas as pl
from jax.experimental.pallas import tpu as pltpu
from jax.experimental.pallas import tpu_sc as plsc

L = 256
PERMS = ((0, 1, 2, 3), (2, 0, 3, 1), (3, 2, 1, 0), (1, 3, 0, 2))

NPLANES = 24
H = 512
W = 512
PW = 536
BAND = 32
NTASKS = NPLANES * (H // BAND)
NTILES = 32
NK = 520
FAMS = ((0, 0, 0, 1), (0, 1, 1, 1), (0, 0, 1, 1), (0, 1, 1, 0))
GCHUNKS = ((0, 128), (128, 128), (256, 128), (384, 128), (512, 8))


FAM_OFF = (0, 65536, 131072, 196608)


def _sc_body(pimg, tbl, out, img_band, idx_scr, g, acc, stage,
             sem_g0, sem_g1, sem_o0, sem_o1):
    i32 = jnp.int32
    wid = lax.axis_index("s") * 2 + lax.axis_index("c")
    iota = lax.broadcasted_iota(i32, (16,), 0)
    i2 = iota >> 1
    i4 = iota >> 2
    c2 = iota & 1
    c4 = iota & 3
    pa_flat = (i2 << 2) + c2

    sem_g = (sem_g0, sem_g1)
    sem_o = (sem_o0, sem_o1)

    def _src(f):
        return tbl

    def _idx_slice(gb, f, ci, n):
        return (idx_scr.at[gb, f, ci] if n == 128
                else idx_scr.at[gb, f, ci, pl.ds(0, n)])

    def idx_fire20(i, gb):
        ro1 = (i + 1) * PW
        ro2 = (i + 2) * PW

        def chunk_body(c, _):
            k0 = c * 16
            colk = iota + k0
            v00 = plsc.load_gather(img_band, [colk + ro1])
            v01 = plsc.load_gather(img_band, [colk + (ro1 + 1)])
            v10 = plsc.load_gather(img_band, [colk + ro2])
            v11 = plsc.load_gather(img_band, [colk + (ro2 + 1)])
            jj = k0 >> 7
            off = k0 & 127
            vv = (v00, v01, v10, v11)
            src = ((0, 1), (1, 3), (0, 3), (1, 2))
            for f, (ai, bi) in enumerate(src):
                idx_scr[gb, f, jj, pl.ds(off, 16)] = \
                    (vv[ai] << 8) + vv[bi] + FAM_OFF[f]
            return _

        lax.fori_loop(0, (NK + 15) // 16, chunk_body, 0)
        for f in range(4):
            for ci, (st, n) in enumerate(GCHUNKS):
                pltpu.async_copy(_src(f).at[_idx_slice(gb, f, ci, n)],
                                 g.at[gb, f, pl.ds(st, n)], sem_g[gb])

    def drain20(gb):
        for f in range(4):
            for ci, (st, n) in enumerate(GCHUNKS):
                pltpu.make_async_copy(_src(f).at[_idx_slice(gb, f, ci, n)],
                                      g.at[gb, f, pl.ds(st, n)],
                                      sem_g[gb]).wait()

    def next_part(gb, ap):
        def body(vv, _):
            rq = i4 + vv * 4
            cc = c4 + 4
            s = plsc.load_gather(g.at[gb, 1], [rq, cc])
            s = s + plsc.load_gather(g.at[gb, 2], [rq, cc])
            s = s + plsc.load_gather(g.at[gb, 3], [rq + 1, cc])
            acc[ap, pl.ds(vv * 16, 16)] = s
            return _

        lax.fori_loop(0, 128, body, 0)

    def emit_cur(gb, ap, plane_row, ob):
        for half in range(2):
            ch = c2 + 2 * half
            ch4 = ch + 4

            def body(v, _, half=half, ch=ch, ch4=ch4):
                rq = i2 + v * 8
                s = plsc.load_gather(acc.at[ap], [pa_flat + (v * 32 + 2 * half)])
                s = s + plsc.load_gather(g.at[gb, 0], [rq + 1, ch])
                s = s + plsc.load_gather(g.at[gb, 0], [rq, ch4])
                s = s + plsc.load_gather(g.at[gb, 1], [rq, ch])
                s = s + plsc.load_gather(g.at[gb, 2], [rq + 1, ch])
                s = s + plsc.load_gather(g.at[gb, 3], [rq, ch])
                stage[ob, pl.ds(half * 1024 + v * 16, 16)] = s
                return _

            lax.fori_loop(0, 64, body, 0)
        pltpu.async_copy(stage.at[ob], out.at[pl.ds(plane_row * 2048, 2048)],
                         sem_o[ob])

    def wait_out(ob):
        pltpu.make_async_copy(out.at[pl.ds(0, 2048)], stage.at[ob],
                              sem_o[ob]).wait()

    def band_body(k, _):
        t = wid + NTILES * k
        plane = t // (H // BAND)
        r0 = (t % (H // BAND)) * BAND
        base = (plane * (H + 2) + r0) * PW
        hs = [pltpu.async_copy(pimg.at[pl.ds(base + rr * PW, PW)],
                               img_band.at[pl.ds(rr * PW, PW)], sem_g0)
              for rr in range(BAND + 2)]
        for h in hs:
            h.wait()

        idx_fire20(-1, 1)
        drain20(1)
        next_part(1, 0)
        idx_fire20(0, 0)

        def body2(u, _):
            i0 = u * 2
            idx_fire20(i0 + 1, 1)
            drain20(0)

            @pl.when(u > 0)
            def _w0():
                wait_out(0)
            emit_cur(0, 0, plane * H + r0 + i0, 0)
            next_part(0, 1)
            @pl.when(u < (BAND // 2) - 1)
            def _f0():
                idx_fire20(i0 + 2, 0)
            drain20(1)

            @pl.when(u > 0)
            def _w1():
                wait_out(1)
            emit_cur(1, 1, plane * H + r0 + i0 + 1, 1)
            next_part(1, 0)
            return _

        lax.fori_loop(0, BAND // 2, body2, 0)
        wait_out(0)
        wait_out(1)
        return _

    lax.fori_loop(0, NTASKS // NTILES, band_body, 0)


@jax.jit
def kernel(img_lr, h_weight, d_weight):
    B, C, _, _ = img_lr.shape
    pimg = img_lr.reshape(B * C, H, W).astype(jnp.int32)
    pimg = jnp.pad(pimg, ((0, 0), (1, 1), (1, PW - W - 1)), mode="edge")
    pimg = pimg.reshape(-1)

    def swap_ab(x):
        return x.reshape(L, L, 4).transpose(1, 0, 2).reshape(L * L, 4)

    parts = []
    for w, r in ((h_weight, 0), (h_weight, 1), (d_weight, 0), (d_weight, 1)):
        first = 0.5 * w[:, jnp.array(PERMS[r])]
        second = swap_ab(0.5 * w[:, jnp.array(PERMS[r + 2])])
        parts.append(jnp.concatenate([first, second], axis=1))
    tbl = jnp.concatenate(parts, axis=0)

    mesh = plsc.VectorSubcoreMesh(core_axis_name="c", subcore_axis_name="s")
    run = pl.kernel(
        _sc_body,
        mesh=mesh,
        out_type=jax.ShapeDtypeStruct((NPLANES * H * 2 * 2 * W,), jnp.float32),
        scratch_types=[
            pltpu.VMEM(((BAND + 2) * PW,), jnp.int32),
            pltpu.VMEM((2, 4, 5, 128), jnp.int32),
            pltpu.VMEM((2, 4, NK, 8), jnp.float32),
            pltpu.VMEM((2, 4 * W), jnp.float32),
            pltpu.VMEM((2, 2 * 2 * W), jnp.float32),
            pltpu.SemaphoreType.DMA,
            pltpu.SemaphoreType.DMA,
            pltpu.SemaphoreType.DMA,
            pltpu.SemaphoreType.DMA,
        ],
        compiler_params=pltpu.CompilerParams(
            needs_layout_passes=False, use_tc_tiling_on_sc=False),
    )
    out = run(pimg, tbl)
    return out.reshape(B, C, 2 * H, 2 * W)


# speedup vs baseline: 1.0325x; 1.0325x over previous
import jax
import jax.numpy as jnp
from jax import lax
from jax.experimental import pallas as pl
from jax.experimental.pallas import tpu as pltpu
from jax.experimental.pallas import tpu_sc as plsc

L = 256
PERMS = ((0, 1, 2, 3), (2, 0, 3, 1), (3, 2, 1, 0), (1, 3, 0, 2))

NPLANES = 24
H = 512
W = 512
PW = 536
BAND = 64
NTASKS = NPLANES * (H // BAND)
NTILES = 32
NK = 520
FAMS = ((0, 0, 0, 1), (0, 1, 1, 1), (0, 0, 1, 1), (0, 1, 1, 0))
GCHUNKS = ((0, 128), (128, 128), (256, 128), (384, 128), (512, 8))


FAM_OFF = (0, 65536, 131072, 196608)


def _sc_body(pimg, tbl, out, img_band, idx_scr, g, acc, stage,
             sem_g0, sem_g1, sem_o0, sem_o1):
    i32 = jnp.int32
    wid = lax.axis_index("s") * 2 + lax.axis_index("c")
    iota = lax.broadcasted_iota(i32, (16,), 0)
    i2 = iota >> 1
    i4 = iota >> 2
    c2 = iota & 1
    c4 = iota & 3
    pa_flat = (i2 << 2) + c2

    sem_g = (sem_g0, sem_g1)
    sem_o = (sem_o0, sem_o1)

    def _src(f):
        return tbl

    def _idx_slice(gb, f, ci, n):
        return (idx_scr.at[gb, f, ci] if n == 128
                else idx_scr.at[gb, f, ci, pl.ds(0, n)])

    def idx_fire20(i, gb):
        ro1 = (i + 1) * PW
        ro2 = (i + 2) * PW

        def chunk_body(c, _):
            k0 = c * 16
            colk = iota + k0
            v00 = plsc.load_gather(img_band, [colk + ro1])
            v01 = plsc.load_gather(img_band, [colk + (ro1 + 1)])
            v10 = plsc.load_gather(img_band, [colk + ro2])
            v11 = plsc.load_gather(img_band, [colk + (ro2 + 1)])
            jj = k0 >> 7
            off = k0 & 127
            vv = (v00, v01, v10, v11)
            src = ((0, 1), (1, 3), (0, 3), (1, 2))
            for f, (ai, bi) in enumerate(src):
                idx_scr[gb, f, jj, pl.ds(off, 16)] = \
                    (vv[ai] << 8) + vv[bi] + FAM_OFF[f]
            return _

        lax.fori_loop(0, (NK + 15) // 16, chunk_body, 0)
        for f in range(4):
            for ci, (st, n) in enumerate(GCHUNKS):
                pltpu.async_copy(_src(f).at[_idx_slice(gb, f, ci, n)],
                                 g.at[gb, f, pl.ds(st, n)], sem_g[gb])

    def drain20(gb):
        for f in range(4):
            for ci, (st, n) in enumerate(GCHUNKS):
                pltpu.make_async_copy(_src(f).at[_idx_slice(gb, f, ci, n)],
                                      g.at[gb, f, pl.ds(st, n)],
                                      sem_g[gb]).wait()

    def next_part(gb, ap):
        def body(vv, _):
            rq = i4 + vv * 4
            cc = c4 + 4
            s = plsc.load_gather(g.at[gb, 1], [rq, cc])
            s = s + plsc.load_gather(g.at[gb, 2], [rq, cc])
            s = s + plsc.load_gather(g.at[gb, 3], [rq + 1, cc])
            acc[ap, pl.ds(vv * 16, 16)] = s
            return _

        lax.fori_loop(0, 128, body, 0)

    def emit_cur(gb, ap, plane_row, ob):
        for half in range(2):
            ch = c2 + 2 * half
            ch4 = ch + 4

            def body(v, _, half=half, ch=ch, ch4=ch4):
                rq = i2 + v * 8
                s = plsc.load_gather(acc.at[ap], [pa_flat + (v * 32 + 2 * half)])
                s = s + plsc.load_gather(g.at[gb, 0], [rq + 1, ch])
                s = s + plsc.load_gather(g.at[gb, 0], [rq, ch4])
                s = s + plsc.load_gather(g.at[gb, 1], [rq, ch])
                s = s + plsc.load_gather(g.at[gb, 2], [rq + 1, ch])
                s = s + plsc.load_gather(g.at[gb, 3], [rq, ch])
                stage[ob, pl.ds(half * 1024 + v * 16, 16)] = s
                return _

            lax.fori_loop(0, 64, body, 0)
        pltpu.async_copy(stage.at[ob], out.at[pl.ds(plane_row * 2048, 2048)],
                         sem_o[ob])

    def wait_out(ob):
        pltpu.make_async_copy(out.at[pl.ds(0, 2048)], stage.at[ob],
                              sem_o[ob]).wait()

    def band_body(k, _):
        t = wid + NTILES * k
        plane = t // (H // BAND)
        r0 = (t % (H // BAND)) * BAND
        base = (plane * (H + 2) + r0) * PW
        hs = [pltpu.async_copy(pimg.at[pl.ds(base + rr * PW, PW)],
                               img_band.at[pl.ds(rr * PW, PW)], sem_g0)
              for rr in range(BAND + 2)]
        for h in hs:
            h.wait()

        idx_fire20(-1, 1)
        drain20(1)
        next_part(1, 0)
        idx_fire20(0, 0)

        def body2(u, _):
            i0 = u * 2
            idx_fire20(i0 + 1, 1)
            drain20(0)

            @pl.when(u > 0)
            def _w0():
                wait_out(0)
            emit_cur(0, 0, plane * H + r0 + i0, 0)
            next_part(0, 1)
            @pl.when(u < (BAND // 2) - 1)
            def _f0():
                idx_fire20(i0 + 2, 0)
            drain20(1)

            @pl.when(u > 0)
            def _w1():
                wait_out(1)
            emit_cur(1, 1, plane * H + r0 + i0 + 1, 1)
            next_part(1, 0)
            return _

        lax.fori_loop(0, BAND // 2, body2, 0)
        wait_out(0)
        wait_out(1)
        return _

    lax.fori_loop(0, NTASKS // NTILES, band_body, 0)


@jax.jit
def kernel(img_lr, h_weight, d_weight):
    B, C, _, _ = img_lr.shape
    pimg = img_lr.reshape(B * C, H, W).astype(jnp.int32)
    pimg = jnp.pad(pimg, ((0, 0), (1, 1), (1, PW - W - 1)), mode="edge")
    pimg = pimg.reshape(-1)

    def swap_ab(x):
        return x.reshape(L, L, 4).transpose(1, 0, 2).reshape(L * L, 4)

    parts = []
    for w, r in ((h_weight, 0), (h_weight, 1), (d_weight, 0), (d_weight, 1)):
        first = 0.5 * w[:, jnp.array(PERMS[r])]
        second = swap_ab(0.5 * w[:, jnp.array(PERMS[r + 2])])
        parts.append(jnp.concatenate([first, second], axis=1))
    tbl = jnp.concatenate(parts, axis=0)

    mesh = plsc.VectorSubcoreMesh(core_axis_name="c", subcore_axis_name="s")
    run = pl.kernel(
        _sc_body,
        mesh=mesh,
        out_type=jax.ShapeDtypeStruct((NPLANES * H * 2 * 2 * W,), jnp.float32),
        scratch_types=[
            pltpu.VMEM(((BAND + 2) * PW,), jnp.int32),
            pltpu.VMEM((2, 4, 5, 128), jnp.int32),
            pltpu.VMEM((2, 4, NK, 8), jnp.float32),
            pltpu.VMEM((2, 4 * W), jnp.float32),
            pltpu.VMEM((2, 2 * 2 * W), jnp.float32),
            pltpu.SemaphoreType.DMA,
            pltpu.SemaphoreType.DMA,
            pltpu.SemaphoreType.DMA,
            pltpu.SemaphoreType.DMA,
        ],
        compiler_params=pltpu.CompilerParams(
            needs_layout_passes=False, use_tc_tiling_on_sc=False),
    )
    out = run(pimg, tbl)
    return out.reshape(B, C, 2 * H, 2 * W)


# speedup vs baseline: 1.0500x; 1.0170x over previous
import jax
import jax.numpy as jnp
from jax import lax
from jax.experimental import pallas as pl
from jax.experimental.pallas import tpu as pltpu
from jax.experimental.pallas import tpu_sc as plsc

L = 256
PERMS = ((0, 1, 2, 3), (2, 0, 3, 1), (3, 2, 1, 0), (1, 3, 0, 2))

NPLANES = 24
H = 512
W = 512
PW = 536
BAND = 128
NTASKS = NPLANES * (H // BAND)
NTILES = 32
NK = 520
FAMS = ((0, 0, 0, 1), (0, 1, 1, 1), (0, 0, 1, 1), (0, 1, 1, 0))
GCHUNKS = ((0, 128), (128, 128), (256, 128), (384, 128), (512, 8))


FAM_OFF = (0, 65536, 131072, 196608)


def _sc_body(pimg, tbl, out, img_band, idx_scr, g, acc, stage,
             sem_g0, sem_g1, sem_o0, sem_o1):
    i32 = jnp.int32
    wid = lax.axis_index("s") * 2 + lax.axis_index("c")
    iota = lax.broadcasted_iota(i32, (16,), 0)
    i2 = iota >> 1
    i4 = iota >> 2
    c2 = iota & 1
    c4 = iota & 3
    pa_flat = (i2 << 2) + c2

    sem_g = (sem_g0, sem_g1)
    sem_o = (sem_o0, sem_o1)

    def _src(f):
        return tbl

    def _idx_slice(gb, f, ci, n):
        return (idx_scr.at[gb, f, ci] if n == 128
                else idx_scr.at[gb, f, ci, pl.ds(0, n)])

    def idx_fire20(i, gb):
        ro1 = (i + 1) * PW
        ro2 = (i + 2) * PW

        def chunk_body(c, _):
            k0 = c * 16
            colk = iota + k0
            v00 = plsc.load_gather(img_band, [colk + ro1])
            v01 = plsc.load_gather(img_band, [colk + (ro1 + 1)])
            v10 = plsc.load_gather(img_band, [colk + ro2])
            v11 = plsc.load_gather(img_band, [colk + (ro2 + 1)])
            jj = k0 >> 7
            off = k0 & 127
            vv = (v00, v01, v10, v11)
            src = ((0, 1), (1, 3), (0, 3), (1, 2))
            for f, (ai, bi) in enumerate(src):
                idx_scr[gb, f, jj, pl.ds(off, 16)] = \
                    (vv[ai] << 8) + vv[bi] + FAM_OFF[f]
            return _

        lax.fori_loop(0, (NK + 15) // 16, chunk_body, 0)
        for f in range(4):
            for ci, (st, n) in enumerate(GCHUNKS):
                pltpu.async_copy(_src(f).at[_idx_slice(gb, f, ci, n)],
                                 g.at[gb, f, pl.ds(st, n)], sem_g[gb])

    def drain20(gb):
        for f in range(4):
            for ci, (st, n) in enumerate(GCHUNKS):
                pltpu.make_async_copy(_src(f).at[_idx_slice(gb, f, ci, n)],
                                      g.at[gb, f, pl.ds(st, n)],
                                      sem_g[gb]).wait()

    def next_part(gb, ap):
        def body(vv, _):
            rq = i4 + vv * 4
            cc = c4 + 4
            s = plsc.load_gather(g.at[gb, 1], [rq, cc])
            s = s + plsc.load_gather(g.at[gb, 2], [rq, cc])
            s = s + plsc.load_gather(g.at[gb, 3], [rq + 1, cc])
            acc[ap, pl.ds(vv * 16, 16)] = s
            return _

        lax.fori_loop(0, 128, body, 0)

    def emit_cur(gb, ap, plane_row, ob):
        for half in range(2):
            ch = c2 + 2 * half
            ch4 = ch + 4

            def body(v, _, half=half, ch=ch, ch4=ch4):
                rq = i2 + v * 8
                s = plsc.load_gather(acc.at[ap], [pa_flat + (v * 32 + 2 * half)])
                s = s + plsc.load_gather(g.at[gb, 0], [rq + 1, ch])
                s = s + plsc.load_gather(g.at[gb, 0], [rq, ch4])
                s = s + plsc.load_gather(g.at[gb, 1], [rq, ch])
                s = s + plsc.load_gather(g.at[gb, 2], [rq + 1, ch])
                s = s + plsc.load_gather(g.at[gb, 3], [rq, ch])
                stage[ob, pl.ds(half * 1024 + v * 16, 16)] = s
                return _

            lax.fori_loop(0, 64, body, 0)
        pltpu.async_copy(stage.at[ob], out.at[pl.ds(plane_row * 2048, 2048)],
                         sem_o[ob])

    def wait_out(ob):
        pltpu.make_async_copy(out.at[pl.ds(0, 2048)], stage.at[ob],
                              sem_o[ob]).wait()

    def band_body(k, _):
        t = wid + NTILES * k
        plane = t // (H // BAND)
        r0 = (t % (H // BAND)) * BAND
        base = (plane * (H + 2) + r0) * PW
        hs = [pltpu.async_copy(pimg.at[pl.ds(base + rr * PW, PW)],
                               img_band.at[pl.ds(rr * PW, PW)], sem_g0)
              for rr in range(BAND + 2)]
        for h in hs:
            h.wait()

        idx_fire20(-1, 1)
        drain20(1)
        next_part(1, 0)
        idx_fire20(0, 0)

        def body2(u, _):
            i0 = u * 2
            idx_fire20(i0 + 1, 1)
            drain20(0)

            @pl.when(u > 0)
            def _w0():
                wait_out(0)
            emit_cur(0, 0, plane * H + r0 + i0, 0)
            next_part(0, 1)
            @pl.when(u < (BAND // 2) - 1)
            def _f0():
                idx_fire20(i0 + 2, 0)
            drain20(1)

            @pl.when(u > 0)
            def _w1():
                wait_out(1)
            emit_cur(1, 1, plane * H + r0 + i0 + 1, 1)
            next_part(1, 0)
            return _

        lax.fori_loop(0, BAND // 2, body2, 0)
        wait_out(0)
        wait_out(1)
        return _

    lax.fori_loop(0, NTASKS // NTILES, band_body, 0)


@jax.jit
def kernel(img_lr, h_weight, d_weight):
    B, C, _, _ = img_lr.shape
    pimg = img_lr.reshape(B * C, H, W).astype(jnp.int32)
    pimg = jnp.pad(pimg, ((0, 0), (1, 1), (1, PW - W - 1)), mode="edge")
    pimg = pimg.reshape(-1)

    def swap_ab(x):
        return x.reshape(L, L, 4).transpose(1, 0, 2).reshape(L * L, 4)

    parts = []
    for w, r in ((h_weight, 0), (h_weight, 1), (d_weight, 0), (d_weight, 1)):
        first = 0.5 * w[:, jnp.array(PERMS[r])]
        second = swap_ab(0.5 * w[:, jnp.array(PERMS[r + 2])])
        parts.append(jnp.concatenate([first, second], axis=1))
    tbl = jnp.concatenate(parts, axis=0)

    mesh = plsc.VectorSubcoreMesh(core_axis_name="c", subcore_axis_name="s")
    run = pl.kernel(
        _sc_body,
        mesh=mesh,
        out_type=jax.ShapeDtypeStruct((NPLANES * H * 2 * 2 * W,), jnp.float32),
        scratch_types=[
            pltpu.VMEM(((BAND + 2) * PW,), jnp.int32),
            pltpu.VMEM((2, 4, 5, 128), jnp.int32),
            pltpu.VMEM((2, 4, NK, 8), jnp.float32),
            pltpu.VMEM((2, 4 * W), jnp.float32),
            pltpu.VMEM((2, 2 * 2 * W), jnp.float32),
            pltpu.SemaphoreType.DMA,
            pltpu.SemaphoreType.DMA,
            pltpu.SemaphoreType.DMA,
            pltpu.SemaphoreType.DMA,
        ],
        compiler_params=pltpu.CompilerParams(
            needs_layout_passes=False, use_tc_tiling_on_sc=False),
    )
    out = run(pimg, tbl)
    return out.reshape(B, C, 2 * H, 2 * W)
